# baseline (speedup 1.0000x reference)
import functools

import jax
import jax.numpy as jnp
from jax import lax
from jax.experimental import pallas as pl
from jax.experimental.pallas import tpu as pltpu
from jax.experimental.pallas import tpu_sc as plsc

VOCAB = 1000000
EMBED_DIM = 32
BATCH = 16384
HIST_LEN = 50

NC = 2
NS = 16
NW = NC * NS
B_PER_W = BATCH // NW
NBLK = B_PER_W // 128
NGF = EMBED_DIM // 8
NPAIRS = HIST_LEN // 2

_mesh = plsc.VectorSubcoreMesh(core_axis_name="c", subcore_axis_name="s")


NBLKS_TOT = VOCAB // 128
BLK_PER_W = NBLKS_TOT // NW
KCH = 4
NCH_W = BLK_PER_W // KCH
A_NPAIRS = (NCH_W - 1) // 2


@functools.partial(
    pl.kernel,
    out_type=jax.ShapeDtypeStruct((VOCAB // 4, 128), jnp.float32),
    mesh=_mesh,
    scratch_types=[
        pltpu.VMEM((32, KCH * 128), jnp.float32),
        pltpu.VMEM((32, KCH * 128), jnp.float32),
        pltpu.VMEM((KCH * 32, 128), jnp.float32),
        pltpu.VMEM((KCH * 32, 128), jnp.float32),
        pltpu.VMEM((32, 64), jnp.float32),
        pltpu.SemaphoreType.DMA,
        pltpu.SemaphoreType.DMA,
        pltpu.SemaphoreType.DMA,
        pltpu.SemaphoreType.DMA,
    ],
    compiler_params=pltpu.CompilerParams(use_tc_tiling_on_sc=True,
                                         needs_layout_passes=False),
)
def _table_relayout(tableT_hbm, out_hbm, inb0, inb1, ob0, ob1, tb,
                    isem0, isem1, osem0, osem1):
    wid = lax.axis_index("s") * NC + lax.axis_index("c")
    blk0 = pl.multiple_of(wid * BLK_PER_W, KCH)

    lanes = lax.iota(jnp.int32, 16)
    row4 = lanes // 4
    col4 = (lanes % 4) * 32

    inb = (inb0, inb1)
    ob = (ob0, ob1)
    isem = (isem0, isem1)
    osem = (osem0, osem1)

    def fire_in(c, slot):
        r0 = (blk0 + c * KCH) * 128
        for g in range(NGF):
            pltpu.async_copy(
                tableT_hbm.at[pl.ds(g * 8, 8), pl.ds(r0, KCH * 128)],
                inb[slot].at[pl.ds(g * 8, 8)],
                isem[slot],
            )

    def drain_in(slot):
        for g in range(NGF):
            pltpu.make_async_copy(
                tableT_hbm.at[pl.ds(0, 8), pl.ds(0, KCH * 128)],
                inb[slot].at[pl.ds(g * 8, 8)],
                isem[slot],
            ).wait()

    def transpose_tiles(slot):
        src = inb[slot]
        dst = ob[slot]

        @plsc.parallel_loop(0, 8, unroll=2)
        def _(m):
            m4 = m * 4
            for b in range(KCH):
                ridx = row4 + (b * 32 + m4)
                for g in range(NGF):
                    for c8 in range(8):
                        v = src[g * 8 + c8, pl.ds(b * 128 + m * 16, 16)]
                        plsc.store_scatter(dst, [ridx, col4 + (8 * g + c8)], v)

    def fire_out(c, slot):
        q0 = (blk0 + c * KCH) * 32
        pltpu.async_copy(ob[slot], out_hbm.at[pl.ds(q0, KCH * 32)], osem[slot])

    def drain_out(slot):
        pltpu.make_async_copy(
            ob[slot], out_hbm.at[pl.ds(0, KCH * 32)], osem[slot]
        ).wait()

    fire_in(0, 0)

    def pair_body(p, carry):
        c0 = p * 2
        c1 = c0 + 1

        @pl.when(p > 0)
        def _():
            drain_out(1)

        fire_in(c1, 1)

        drain_in(0)
        transpose_tiles(0)
        fire_out(c0, 0)

        drain_out(0)
        fire_in(c0 + 2, 0)

        drain_in(1)
        transpose_tiles(1)
        fire_out(c1, 1)
        return carry

    lax.fori_loop(0, A_NPAIRS, pair_body, 0)
    drain_out(1)
    drain_in(0)
    transpose_tiles(0)
    fire_out(NCH_W - 1, 0)
    drain_out(0)

    @pl.when(wid == 0)
    def _():
        for g in range(NGF):
            pltpu.async_copy(
                tableT_hbm.at[pl.ds(g * 8, 8),
                              pl.ds(NW * BLK_PER_W * 128, KCH * 128)],
                inb[0].at[pl.ds(g * 8, 8)],
                isem[0],
            )
        drain_in(0)
        transpose_tiles(0)
        pltpu.async_copy(
            ob[0], out_hbm.at[pl.ds(NW * BLK_PER_W * 32, KCH * 32)], osem[0]
        )
        for g in range(NGF):
            pltpu.async_copy(
                tableT_hbm.at[pl.ds(g * 8, 8), pl.ds(NBLKS_TOT * 128, 64)],
                tb.at[pl.ds(g * 8, 8)],
                isem[0],
            ).wait()
        drain_out(0)
        for m in range(4):
            ridx = row4 + m * 4
            for g in range(NGF):
                for c8 in range(8):
                    v = tb[g * 8 + c8, pl.ds(m * 16, 16)]
                    plsc.store_scatter(ob[0], [ridx, col4 + (8 * g + c8)], v)
        pltpu.async_copy(
            ob[0].at[pl.ds(0, 16)],
            out_hbm.at[pl.ds(NBLKS_TOT * 32, 16)],
            osem[0],
        )
        pltpu.make_async_copy(
            ob[0].at[pl.ds(0, 16)], out_hbm.at[pl.ds(0, 16)], osem[0]
        ).wait()


@functools.partial(
    pl.kernel,
    out_type=jax.ShapeDtypeStruct((HIST_LEN, NGF, NW, NBLK * 8 * 128),
                                  jnp.float32),
    mesh=_mesh,
    scratch_types=[
        pltpu.VMEM((B_PER_W, HIST_LEN), jnp.int32),
        pltpu.VMEM((HIST_LEN * B_PER_W,), jnp.int32),
        pltpu.VMEM((B_PER_W, EMBED_DIM), jnp.float32),
        pltpu.VMEM((B_PER_W, EMBED_DIM), jnp.float32),
        pltpu.VMEM((NGF * NBLK * 8 * 128,), jnp.float32),
        pltpu.VMEM((NGF * NBLK * 8 * 128,), jnp.float32),
        pltpu.SemaphoreType.DMA,
        pltpu.SemaphoreType.DMA,
        pltpu.SemaphoreType.DMA,
        pltpu.SemaphoreType.DMA,
        pltpu.SemaphoreType.DMA,
    ],
    compiler_params=pltpu.CompilerParams(use_tc_tiling_on_sc=False,
                                         needs_layout_passes=False),
)
def _emb_gather(title_hbm, table_hbm, out_hbm, idx_b, idx_t, rows0, rows1,
                tiles0, tiles1, isem, gsem0, gsem1, wsem0, wsem1):
    wid = lax.axis_index("s") * NC + lax.axis_index("c")
    base = pl.multiple_of(wid * B_PER_W, B_PER_W)

    lanes = lax.iota(jnp.int32, 16)
    pos_lo = (lanes // 8) * 4096 + (lanes % 8) * 128
    pos_hi = pos_lo + 2 * 4096

    pltpu.async_copy(title_hbm.at[pl.ds(base, B_PER_W)], idx_b, isem).wait()

    @plsc.parallel_loop(0, HIST_LEN, unroll=2)
    def _(t):
        tb = t * B_PER_W
        col = lanes * 0 + t
        for m in range(B_PER_W // 16):
            src = plsc.load_gather(idx_b, [lanes + 16 * m, col])
            idx_t[pl.ds(tb + 16 * m, 16)] = src

    rows = (rows0, rows1)
    tiles = (tiles0, tiles1)
    gsem = (gsem0, gsem1)
    wsem = (wsem0, wsem1)

    def fire_gathers(t, slot):
        for k in range(NBLK):
            pltpu.async_copy(
                table_hbm.at[idx_t.at[pl.ds(t * B_PER_W + k * 128, 128)]],
                rows[slot].at[pl.ds(k * 128, 128)],
                gsem[slot],
            )

    def drain_gathers(slot):
        for k in range(NBLK):
            pltpu.make_async_copy(
                table_hbm.at[pl.ds(0, 128)],
                rows[slot].at[pl.ds(k * 128, 128)],
                gsem[slot],
            ).wait()

    def transpose_rows(slot):
        rv = rows[slot]
        tv = tiles[slot]
        for blk in range(NBLK):
            base_lo = pos_lo + blk * 1024
            base_hi = pos_hi + blk * 1024

            @plsc.parallel_loop(0, 128, unroll=16)
            def _(j, blk=blk, base_lo=base_lo, base_hi=base_hi):
                b = blk * 128 + j
                plsc.store_scatter(tv, [base_lo + j], rv[b, pl.ds(0, 16)])
                plsc.store_scatter(tv, [base_hi + j], rv[b, pl.ds(16, 16)])

    def fire_write(t, slot):
        for g in range(NGF):
            pltpu.async_copy(
                tiles[slot].at[pl.ds(g * 4096, 4096)],
                out_hbm.at[t, g, wid],
                wsem[slot],
            )

    def drain_write(slot):
        for g in range(NGF):
            pltpu.make_async_copy(
                tiles[slot].at[pl.ds(g * 4096, 4096)],
                out_hbm.at[0, 0, 0],
                wsem[slot],
            ).wait()

    fire_gathers(0, 0)

    def pair_body(p, carry):
        t0 = p * 2
        t1 = t0 + 1

        @pl.when(p > 0)
        def _():
            drain_write(1)

        fire_gathers(t1, 1)

        drain_gathers(0)
        transpose_rows(0)
        fire_write(t0, 0)

        @pl.when(p + 1 < NPAIRS)
        def _():
            drain_write(0)
            fire_gathers(t0 + 2, 0)

        drain_gathers(1)
        transpose_rows(1)
        fire_write(t1, 1)
        return carry

    lax.fori_loop(0, NPAIRS, pair_body, 0)
    drain_write(0)
    drain_write(1)


def kernel(title, table):
    t_rm = _table_relayout(table.T).reshape(VOCAB, EMBED_DIM)
    x = _emb_gather(title.astype(jnp.int32), t_rm)
    x = x.reshape(HIST_LEN, NGF, BATCH // 128, 8, 128)
    return x.transpose(2, 4, 0, 1, 3).reshape(BATCH, HIST_LEN, EMBED_DIM)


# speedup vs baseline: 1.0403x; 1.0403x over previous
import functools

import jax
import jax.numpy as jnp
from jax import lax
from jax.experimental import pallas as pl
from jax.experimental.pallas import tpu as pltpu
from jax.experimental.pallas import tpu_sc as plsc

VOCAB = 1000000
EMBED_DIM = 32
BATCH = 16384
HIST_LEN = 50

NC = 2
NS = 16
NW = NC * NS
B_PER_W = BATCH // NW
NBLK = B_PER_W // 128
NGF = EMBED_DIM // 8
NPAIRS = HIST_LEN // 2

_mesh = plsc.VectorSubcoreMesh(core_axis_name="c", subcore_axis_name="s")


@functools.partial(
    pl.kernel,
    out_type=jax.ShapeDtypeStruct((HIST_LEN, NGF, NW, NBLK * 8 * 128),
                                  jnp.float32),
    mesh=_mesh,
    scratch_types=[
        pltpu.VMEM((B_PER_W, HIST_LEN), jnp.int32),
        pltpu.VMEM((HIST_LEN * B_PER_W,), jnp.int32),
        pltpu.VMEM((B_PER_W, EMBED_DIM), jnp.float32),
        pltpu.VMEM((B_PER_W, EMBED_DIM), jnp.float32),
        pltpu.VMEM((NGF * NBLK * 8 * 128,), jnp.float32),
        pltpu.VMEM((NGF * NBLK * 8 * 128,), jnp.float32),
        pltpu.SemaphoreType.DMA,
        pltpu.SemaphoreType.DMA,
        pltpu.SemaphoreType.DMA,
        pltpu.SemaphoreType.DMA,
        pltpu.SemaphoreType.DMA,
    ],
    compiler_params=pltpu.CompilerParams(use_tc_tiling_on_sc=False,
                                         needs_layout_passes=False),
)
def _emb_gather(title_hbm, table_hbm, out_hbm, idx_b, idx_t, rows0, rows1,
                tiles0, tiles1, isem, gsem0, gsem1, wsem0, wsem1):
    wid = lax.axis_index("s") * NC + lax.axis_index("c")
    base = pl.multiple_of(wid * B_PER_W, B_PER_W)

    lanes = lax.iota(jnp.int32, 16)
    pos_lo = (lanes // 8) * 4096 + (lanes % 8) * 128
    pos_hi = pos_lo + 2 * 4096

    pltpu.async_copy(title_hbm.at[pl.ds(base, B_PER_W)], idx_b, isem).wait()

    @plsc.parallel_loop(0, HIST_LEN, unroll=2)
    def _(t):
        tb = t * B_PER_W
        col = lanes * 0 + t
        for m in range(B_PER_W // 16):
            src = plsc.load_gather(idx_b, [lanes + 16 * m, col])
            idx_t[pl.ds(tb + 16 * m, 16)] = src

    rows = (rows0, rows1)
    tiles = (tiles0, tiles1)
    gsem = (gsem0, gsem1)
    wsem = (wsem0, wsem1)

    def fire_gathers(t, slot):
        for k in range(NBLK):
            pltpu.async_copy(
                table_hbm.at[idx_t.at[pl.ds(t * B_PER_W + k * 128, 128)]],
                rows[slot].at[pl.ds(k * 128, 128)],
                gsem[slot],
            )

    def drain_gathers(slot):
        for k in range(NBLK):
            pltpu.make_async_copy(
                table_hbm.at[pl.ds(0, 128)],
                rows[slot].at[pl.ds(k * 128, 128)],
                gsem[slot],
            ).wait()

    def transpose_rows(slot):
        rv = rows[slot]
        tv = tiles[slot]
        for blk in range(NBLK):
            base_lo = pos_lo + blk * 1024
            base_hi = pos_hi + blk * 1024

            @plsc.parallel_loop(0, 128, unroll=16)
            def _(j, blk=blk, base_lo=base_lo, base_hi=base_hi):
                b = blk * 128 + j
                plsc.store_scatter(tv, [base_lo + j], rv[b, pl.ds(0, 16)])
                plsc.store_scatter(tv, [base_hi + j], rv[b, pl.ds(16, 16)])

    def fire_write(t, slot):
        for g in range(NGF):
            pltpu.async_copy(
                tiles[slot].at[pl.ds(g * 4096, 4096)],
                out_hbm.at[t, g, wid],
                wsem[slot],
            )

    def drain_write(slot):
        for g in range(NGF):
            pltpu.make_async_copy(
                tiles[slot].at[pl.ds(g * 4096, 4096)],
                out_hbm.at[0, 0, 0],
                wsem[slot],
            ).wait()

    fire_gathers(0, 0)

    def pair_body(p, carry):
        t0 = p * 2
        t1 = t0 + 1

        @pl.when(p > 0)
        def _():
            drain_write(1)

        fire_gathers(t1, 1)

        drain_gathers(0)
        transpose_rows(0)
        fire_write(t0, 0)

        @pl.when(p + 1 < NPAIRS)
        def _():
            drain_write(0)
            fire_gathers(t0 + 2, 0)

        drain_gathers(1)
        transpose_rows(1)
        fire_write(t1, 1)
        return carry

    lax.fori_loop(0, NPAIRS, pair_body, 0)
    drain_write(0)
    drain_write(1)


def kernel(title, table):
    x = _emb_gather(title.astype(jnp.int32), table)
    x = x.reshape(HIST_LEN, NGF, BATCH // 128, 8, 128)
    return x.transpose(2, 4, 0, 1, 3).reshape(BATCH, HIST_LEN, EMBED_DIM)
